# baseline (speedup 1.0000x reference)
import functools

import jax
import jax.numpy as jnp
import numpy as np
from jax.experimental import pallas as pl
from jax.experimental.pallas import tpu as pltpu

_GROUP_SIZES = (21, 75, 150, 304, 700, 1500, 3000, 4700)
_NUM_CLASSES = int(np.sum(_GROUP_SIZES))
_KMAX = 500
_B = 1024
_RB = 128
_NEG_INF = float("-inf")
_LO_INIT = np.int32(-2139095042)
_HI_INIT = np.int32(2139095040)


def _group_windows():
    offs = np.cumsum([0] + list(_GROUP_SIZES))
    wins = []
    for g, size in enumerate(_GROUP_SIZES):
        off, end = int(offs[g]), int(offs[g + 1])
        ws = (off // 128) * 128
        we = min(((end + 127) // 128) * 128, _NUM_CLASSES)
        wins.append((off, end, ws, we, min(size, _KMAX)))
    return wins


_WINDOWS = _group_windows()


def _key_to_f32(m):
    ti = jnp.where(m < 0, m ^ jnp.int32(0x7FFFFFFF), m)
    return jax.lax.bitcast_convert_type(ti, jnp.float32)


def _rowsum(x, ones):
    return jax.lax.dot_general(
        x, ones, (((1,), (0,)), ((), ())), preferred_element_type=jnp.float32)


def _kl_terms(wsel, e_t, e_s, t, s, m_t, m_s, ones, rows_norm):
    w = wsel * e_t
    z_t = _rowsum(w, ones)
    s_wt = _rowsum(w * t, ones)
    s_ts = _rowsum(w * s, ones)
    z_s = _rowsum(wsel * e_s, ones)
    kl = (s_wt - m_t * z_t - s_ts) / z_t - jnp.log(z_t) + m_s + jnp.log(z_s)
    return jnp.sum(kl) * rows_norm


def _loss_body(s_ref, t_ref, o_ref):
    pid = pl.program_id(0)
    total = jnp.float32(0.0)
    big = []
    for g, (off, end, ws, we, k) in enumerate(_WINDOWS):
        size = end - off
        t = t_ref[:, ws:we]
        s = s_ref[:, ws:we]
        cols = jax.lax.broadcasted_iota(jnp.int32, t.shape, 1) + ws
        mask = (cols >= off) & (cols < end)
        tm = jnp.where(mask, t, _NEG_INF)
        sm = jnp.where(mask, s, _NEG_INF)
        ones = jnp.ones((t.shape[1], 1), jnp.float32)
        norm = jnp.float32(size / float(_NUM_CLASSES) / float(_B))
        if k == size:
            m_t = jnp.max(tm, axis=1, keepdims=True)
            m_s = jnp.max(sm, axis=1, keepdims=True)
            e_t = jnp.exp(tm - m_t)
            e_s = jnp.exp(sm - m_s)
            total = total + _kl_terms(
                jnp.float32(1.0), e_t, e_s, t, s, m_t, m_s, ones, norm)
        else:
            h = t.shape[0] // 2
            for r0, r1 in ((0, h), (h, t.shape[0])):
                big.append((t[r0:r1], s[r0:r1], tm[r0:r1], sm[r0:r1],
                            k, norm, ones))

    nbig = len(big)
    rows = big[0][0].shape[0]
    los = tuple(jnp.full((rows, 1), _LO_INIT, jnp.int32) for _ in range(nbig))
    his = tuple(jnp.full((rows, 1), _HI_INIT, jnp.int32) for _ in range(nbig))

    def body(_, carry):
        los, his = carry
        nlos, nhis = [], []
        for gi in range(nbig):
            lo, hi = los[gi], his[gi]
            mid = (lo >> 1) + (hi >> 1) + ((lo | hi) & 1)
            f_mid = _key_to_f32(mid)
            ind = jnp.where(big[gi][2] >= f_mid, 1.0, 0.0)
            cnt = _rowsum(ind, big[gi][6])
            ge = cnt >= jnp.float32(big[gi][4])
            nlos.append(jnp.where(ge, mid, lo))
            nhis.append(jnp.where(ge, hi, mid - 1))
        return tuple(nlos), tuple(nhis)

    los, his = jax.lax.fori_loop(0, 32, body, (los, his), unroll=8)

    for gi in range(nbig):
        t, s, tm, sm, k, norm, ones = big[gi]
        f_tau = _key_to_f32(los[gi])
        gt01 = jnp.where(tm > f_tau, 1.0, 0.0)
        eq01 = jnp.where(tm == f_tau, 1.0, 0.0)
        cgt = _rowsum(gt01, ones)
        ceq = _rowsum(eq01, ones)
        frac = (jnp.float32(k) - cgt) / ceq
        wsel = gt01 + frac * eq01
        m_t = jnp.max(tm, axis=1, keepdims=True)
        m_s = jnp.max(sm, axis=1, keepdims=True)
        e_t = jnp.exp(tm - m_t)
        e_s = jnp.exp(sm - m_s)
        total = total + _kl_terms(wsel, e_t, e_s, t, s, m_t, m_s, ones, norm)

    o_ref[0, 0] = jnp.where(pid == 0, total, o_ref[0, 0] + total)


@jax.jit
def kernel(logits, logits_teacher, targets):
    del targets
    out = pl.pallas_call(
        _loss_body,
        grid=(_B // _RB,),
        in_specs=[
            pl.BlockSpec((_RB, _NUM_CLASSES), lambda i: (i, 0)),
            pl.BlockSpec((_RB, _NUM_CLASSES), lambda i: (i, 0)),
        ],
        out_specs=pl.BlockSpec(memory_space=pltpu.SMEM),
        out_shape=jax.ShapeDtypeStruct((1, 1), jnp.float32),
    )(logits, logits_teacher)
    return out[0, 0]


# speedup vs baseline: 1.0244x; 1.0244x over previous
import functools

import jax
import jax.numpy as jnp
import numpy as np
from jax.experimental import pallas as pl
from jax.experimental.pallas import tpu as pltpu

_GROUP_SIZES = (21, 75, 150, 304, 700, 1500, 3000, 4700)
_NUM_CLASSES = int(np.sum(_GROUP_SIZES))
_KMAX = 500
_B = 1024
_RB = 128
_NEG_INF = float("-inf")
_LO_INIT = np.int32(-2139095042)
_HI_INIT = np.int32(2139095040)


def _group_windows():
    offs = np.cumsum([0] + list(_GROUP_SIZES))
    wins = []
    for g, size in enumerate(_GROUP_SIZES):
        off, end = int(offs[g]), int(offs[g + 1])
        ws = (off // 128) * 128
        we = min(((end + 127) // 128) * 128, _NUM_CLASSES)
        wins.append((off, end, ws, we, min(size, _KMAX)))
    return wins


_WINDOWS = _group_windows()


def _key_to_f32(m):
    ti = jnp.where(m < 0, m ^ jnp.int32(0x7FFFFFFF), m)
    return jax.lax.bitcast_convert_type(ti, jnp.float32)


def _rowsum(x, ones):
    return jax.lax.dot_general(
        x, ones, (((1,), (0,)), ((), ())), preferred_element_type=jnp.float32)


def _kl_terms(wsel, e_t, e_s, t, s, m_t, m_s, ones, rows_norm):
    w = wsel * e_t
    z_t = _rowsum(w, ones)
    s_wt = _rowsum(w * t, ones)
    s_ts = _rowsum(w * s, ones)
    z_s = _rowsum(wsel * e_s, ones)
    kl = (s_wt - m_t * z_t - s_ts) / z_t - jnp.log(z_t) + m_s + jnp.log(z_s)
    return jnp.sum(kl) * rows_norm


def _loss_body(s_ref, t_ref, madd_ref, o_ref):
    pid = pl.program_id(0)
    total = jnp.float32(0.0)
    big = []
    for g, (off, end, ws, we, k) in enumerate(_WINDOWS):
        size = end - off
        t = t_ref[:, ws:we]
        s = s_ref[:, ws:we]
        madd = madd_ref[g, :, ws:we]
        tm = t + madd
        sm = s + madd
        ones = jnp.ones((t.shape[1], 1), jnp.float32)
        norm = jnp.float32(size / float(_NUM_CLASSES) / float(_B))
        if k == size:
            m_t = jnp.max(tm, axis=1, keepdims=True)
            m_s = jnp.max(sm, axis=1, keepdims=True)
            e_t = jnp.exp(tm - m_t)
            e_s = jnp.exp(sm - m_s)
            total = total + _kl_terms(
                jnp.float32(1.0), e_t, e_s, t, s, m_t, m_s, ones, norm)
        else:
            big.append((t, s, tm, sm, k, norm, ones))

    nbig = len(big)
    rows = big[0][0].shape[0]
    los = tuple(jnp.full((rows, 1), _LO_INIT, jnp.int32) for _ in range(nbig))
    his = tuple(jnp.full((rows, 1), _HI_INIT, jnp.int32) for _ in range(nbig))

    def body(_, carry):
        los, his = carry
        nlos, nhis = [], []
        for gi in range(nbig):
            lo, hi = los[gi], his[gi]
            mid = (lo >> 1) + (hi >> 1) + ((lo | hi) & 1)
            f_mid = _key_to_f32(mid)
            ind = jnp.where(big[gi][2] >= f_mid, 1.0, 0.0)
            cnt = _rowsum(ind, big[gi][6])
            ge = cnt >= jnp.float32(big[gi][4])
            nlos.append(jnp.where(ge, mid, lo))
            nhis.append(jnp.where(ge, hi, mid - 1))
        return tuple(nlos), tuple(nhis)

    los, his = jax.lax.fori_loop(0, 32, body, (los, his), unroll=8)

    for gi in range(nbig):
        t, s, tm, sm, k, norm, ones = big[gi]
        f_tau = _key_to_f32(los[gi])
        gt01 = jnp.where(tm > f_tau, 1.0, 0.0)
        eq01 = jnp.where(tm == f_tau, 1.0, 0.0)
        cgt = _rowsum(gt01, ones)
        ceq = _rowsum(eq01, ones)
        frac = (jnp.float32(k) - cgt) / ceq
        wsel = gt01 + frac * eq01
        m_t = jnp.max(tm, axis=1, keepdims=True)
        m_s = jnp.max(sm, axis=1, keepdims=True)
        e_t = jnp.exp(tm - m_t)
        e_s = jnp.exp(sm - m_s)
        total = total + _kl_terms(wsel, e_t, e_s, t, s, m_t, m_s, ones, norm)

    o_ref[0, 0] = jnp.where(pid == 0, total, o_ref[0, 0] + total)


def _build_madd():
    m = np.full((len(_GROUP_SIZES), 1, _NUM_CLASSES), -np.inf, np.float32)
    offs = np.cumsum([0] + list(_GROUP_SIZES))
    for g in range(len(_GROUP_SIZES)):
        m[g, 0, offs[g]:offs[g + 1]] = 0.0
    return m


_MADD = _build_madd()


@jax.jit
def kernel(logits, logits_teacher, targets):
    del targets
    ng = len(_GROUP_SIZES)
    out = pl.pallas_call(
        _loss_body,
        grid=(_B // _RB,),
        in_specs=[
            pl.BlockSpec((_RB, _NUM_CLASSES), lambda i: (i, 0)),
            pl.BlockSpec((_RB, _NUM_CLASSES), lambda i: (i, 0)),
            pl.BlockSpec((ng, 1, _NUM_CLASSES), lambda i: (0, 0, 0)),
        ],
        out_specs=pl.BlockSpec(memory_space=pltpu.SMEM),
        out_shape=jax.ShapeDtypeStruct((1, 1), jnp.float32),
    )(logits, logits_teacher, jnp.asarray(_MADD))
    return out[0, 0]


# speedup vs baseline: 1.0542x; 1.0291x over previous
import functools

import jax
import jax.numpy as jnp
import numpy as np
from jax.experimental import pallas as pl
from jax.experimental.pallas import tpu as pltpu

_GROUP_SIZES = (21, 75, 150, 304, 700, 1500, 3000, 4700)
_NUM_CLASSES = int(np.sum(_GROUP_SIZES))
_KMAX = 500
_B = 1024
_RB = 128
_NEG_INF = float("-inf")
_LO_INIT = np.int32(-2139095042)
_HI_INIT = np.int32(2139095040)


def _group_windows():
    offs = np.cumsum([0] + list(_GROUP_SIZES))
    wins = []
    for g, size in enumerate(_GROUP_SIZES):
        off, end = int(offs[g]), int(offs[g + 1])
        ws = (off // 128) * 128
        we = min(((end + 127) // 128) * 128, _NUM_CLASSES)
        wins.append((off, end, ws, we, min(size, _KMAX)))
    return wins


_WINDOWS = _group_windows()


def _key_to_f32(m):
    ti = jnp.where(m < 0, m ^ jnp.int32(0x7FFFFFFF), m)
    return jax.lax.bitcast_convert_type(ti, jnp.float32)


def _rowsum(x, ones):
    return jax.lax.dot_general(
        x, ones, (((1,), (0,)), ((), ())), preferred_element_type=jnp.float32)


def _kl_terms(wsel, e_t, e_s, t, s, m_t, m_s, ones, rows_norm):
    w = wsel * e_t
    z_t = _rowsum(w, ones)
    s_wt = _rowsum(w * t, ones)
    s_ts = _rowsum(w * s, ones)
    z_s = _rowsum(wsel * e_s, ones)
    kl = (s_wt - m_t * z_t - s_ts) / z_t - jnp.log(z_t) + m_s + jnp.log(z_s)
    return jnp.sum(kl) * rows_norm


def _loss_body(s_ref, t_ref, o_ref):
    pid = pl.program_id(0)
    total = jnp.float32(0.0)
    big = []
    for g, (off, end, ws, we, k) in enumerate(_WINDOWS):
        size = end - off
        t = t_ref[:, ws:we]
        s = s_ref[:, ws:we]
        cols = jax.lax.broadcasted_iota(jnp.int32, t.shape, 1) + ws
        mask = (cols >= off) & (cols < end)
        tm = jnp.where(mask, t, _NEG_INF)
        sm = jnp.where(mask, s, _NEG_INF)
        ones = jnp.ones((t.shape[1], 1), jnp.float32)
        norm = jnp.float32(size / float(_NUM_CLASSES) / float(_B))
        if k == size:
            m_t = jnp.max(tm, axis=1, keepdims=True)
            m_s = jnp.max(sm, axis=1, keepdims=True)
            e_t = jnp.exp(tm - m_t)
            e_s = jnp.exp(sm - m_s)
            total = total + _kl_terms(
                jnp.float32(1.0), e_t, e_s, t, s, m_t, m_s, ones, norm)
        else:
            big.append((t, s, tm, sm, k, norm, ones))

    nbig = len(big)
    rows = big[0][0].shape[0]
    los = tuple(jnp.full((rows, 1), _LO_INIT, jnp.int32) for _ in range(nbig))
    his = tuple(jnp.full((rows, 1), _HI_INIT, jnp.int32) for _ in range(nbig))

    def body(_, carry):
        los, his = carry
        nlos, nhis = [], []
        for gi in range(nbig):
            lo, hi = los[gi], his[gi]
            mid = (lo >> 1) + (hi >> 1) + ((lo | hi) & 1)
            f_mid = _key_to_f32(mid)
            ind = jnp.where(big[gi][2] >= f_mid, 1.0, 0.0)
            cnt = _rowsum(ind, big[gi][6])
            ge = cnt >= jnp.float32(big[gi][4])
            nlos.append(jnp.where(ge, mid, lo))
            nhis.append(jnp.where(ge, hi, mid - 1))
        return tuple(nlos), tuple(nhis)

    los, his = jax.lax.fori_loop(0, 32, body, (los, his), unroll=16)

    for gi in range(nbig):
        t, s, tm, sm, k, norm, ones = big[gi]
        f_tau = _key_to_f32(los[gi])
        gt01 = jnp.where(tm > f_tau, 1.0, 0.0)
        eq01 = jnp.where(tm == f_tau, 1.0, 0.0)
        cgt = _rowsum(gt01, ones)
        ceq = _rowsum(eq01, ones)
        frac = (jnp.float32(k) - cgt) / ceq
        wsel = gt01 + frac * eq01
        m_t = jnp.max(tm, axis=1, keepdims=True)
        m_s = jnp.max(sm, axis=1, keepdims=True)
        e_t = jnp.exp(tm - m_t)
        e_s = jnp.exp(sm - m_s)
        total = total + _kl_terms(wsel, e_t, e_s, t, s, m_t, m_s, ones, norm)

    o_ref[0, 0] = jnp.where(pid == 0, total, o_ref[0, 0] + total)


@jax.jit
def kernel(logits, logits_teacher, targets):
    del targets
    out = pl.pallas_call(
        _loss_body,
        grid=(_B // _RB,),
        in_specs=[
            pl.BlockSpec((_RB, _NUM_CLASSES), lambda i: (i, 0)),
            pl.BlockSpec((_RB, _NUM_CLASSES), lambda i: (i, 0)),
        ],
        out_specs=pl.BlockSpec(memory_space=pltpu.SMEM),
        out_shape=jax.ShapeDtypeStruct((1, 1), jnp.float32),
    )(logits, logits_teacher)
    return out[0, 0]


# speedup vs baseline: 1.0562x; 1.0019x over previous
import jax
import jax.numpy as jnp
import numpy as np
from jax.experimental import pallas as pl
from jax.experimental.pallas import tpu as pltpu

_GROUP_SIZES = (21, 75, 150, 304, 700, 1500, 3000, 4700)
_NUM_CLASSES = int(np.sum(_GROUP_SIZES))
_KMAX = 500
_B = 1024
_RB = 128
_NEG_INF = float("-inf")
_LO_INIT = np.int32(-2139095042)
_HI_INIT = np.int32(2139095040)


def _group_windows():
    offs = np.cumsum([0] + list(_GROUP_SIZES))
    wins = []
    for g, size in enumerate(_GROUP_SIZES):
        off, end = int(offs[g]), int(offs[g + 1])
        ws = (off // 128) * 128
        we = min(((end + 127) // 128) * 128, _NUM_CLASSES)
        wins.append((off, end, ws, we, min(size, _KMAX)))
    return wins


_WINDOWS = _group_windows()


def _key_to_f32(m):
    ti = jnp.where(m < 0, m ^ jnp.int32(0x7FFFFFFF), m)
    return jax.lax.bitcast_convert_type(ti, jnp.float32)


def _rowsum(x, ones):
    return jax.lax.dot_general(
        x, ones, (((1,), (0,)), ((), ())), preferred_element_type=jnp.float32)


def _kl_terms(wsel, e_t, e_s, t, s, m_t, m_s, ones, rows_norm):
    w = wsel * e_t
    z_t = _rowsum(w, ones)
    s_wt = _rowsum(w * t, ones)
    s_ts = _rowsum(w * s, ones)
    z_s = _rowsum(wsel * e_s, ones)
    kl = (s_wt - m_t * z_t - s_ts) / z_t - jnp.log(z_t) + m_s + jnp.log(z_s)
    return jnp.sum(kl) * rows_norm


def _loss_body(s_ref, t_ref, o_ref):
    pid = pl.program_id(0)
    total = jnp.float32(0.0)
    big = []
    for g, (off, end, ws, we, k) in enumerate(_WINDOWS):
        size = end - off
        t = t_ref[:, ws:we]
        s = s_ref[:, ws:we]
        cols = jax.lax.broadcasted_iota(jnp.int32, t.shape, 1) + ws
        mask = (cols >= off) & (cols < end)
        tm = jnp.where(mask, t, _NEG_INF)
        sm = jnp.where(mask, s, _NEG_INF)
        ones = jnp.ones((t.shape[1], 1), jnp.float32)
        norm = jnp.float32(size / float(_NUM_CLASSES) / float(_B))
        if k == size:
            m_t = jnp.max(tm, axis=1, keepdims=True)
            m_s = jnp.max(sm, axis=1, keepdims=True)
            e_t = jnp.exp(tm - m_t)
            e_s = jnp.exp(sm - m_s)
            total = total + _kl_terms(
                jnp.float32(1.0), e_t, e_s, t, s, m_t, m_s, ones, norm)
        else:
            big.append((t, s, tm, sm, k, norm, ones))

    nbig = len(big)
    rows = big[0][0].shape[0]
    los = tuple(jnp.full((rows, 1), _LO_INIT, jnp.int32) for _ in range(nbig))
    his = tuple(jnp.full((rows, 1), _HI_INIT, jnp.int32) for _ in range(nbig))

    def body(_, carry):
        los, his = carry
        nlos, nhis = [], []
        for gi in range(nbig):
            lo, hi = los[gi], his[gi]
            mid = (lo >> 1) + (hi >> 1) + ((lo | hi) & 1)
            f_mid = _key_to_f32(mid)
            ind = jnp.where(big[gi][2] >= f_mid, 1.0, 0.0)
            cnt = _rowsum(ind, big[gi][6])
            ge = cnt >= jnp.float32(big[gi][4])
            nlos.append(jnp.where(ge, mid, lo))
            nhis.append(jnp.where(ge, hi, mid - 1))
        return tuple(nlos), tuple(nhis)

    los, his = jax.lax.fori_loop(0, 32, body, (los, his), unroll=16)

    for gi in range(nbig):
        t, s, tm, sm, k, norm, ones = big[gi]
        f_tau = _key_to_f32(los[gi])
        gt01 = jnp.where(tm > f_tau, 1.0, 0.0)
        eq01 = jnp.where(tm == f_tau, 1.0, 0.0)
        cgt = _rowsum(gt01, ones)
        ceq = _rowsum(eq01, ones)
        frac = (jnp.float32(k) - cgt) / ceq
        wsel = gt01 + frac * eq01
        m_t = jnp.max(tm, axis=1, keepdims=True)
        m_s = jnp.max(sm, axis=1, keepdims=True)
        e_t = jnp.exp(tm - m_t)
        e_s = jnp.exp(sm - m_s)
        total = total + _kl_terms(wsel, e_t, e_s, t, s, m_t, m_s, ones, norm)

    o_ref[0, 0] = jnp.where(pid == 0, total, o_ref[0, 0] + total)


@jax.jit
def kernel(logits, logits_teacher, targets):
    del targets
    out = pl.pallas_call(
        _loss_body,
        grid=(_B // _RB,),
        in_specs=[
            pl.BlockSpec((_RB, _NUM_CLASSES), lambda i: (i, 0)),
            pl.BlockSpec((_RB, _NUM_CLASSES), lambda i: (i, 0)),
        ],
        out_specs=pl.BlockSpec(memory_space=pltpu.SMEM),
        out_shape=jax.ShapeDtypeStruct((1, 1), jnp.float32),
    )(logits, logits_teacher)
    return out[0, 0]


# speedup vs baseline: 1.0938x; 1.0356x over previous
import jax
import jax.numpy as jnp
import numpy as np
from jax.experimental import pallas as pl
from jax.experimental.pallas import tpu as pltpu

_GROUP_SIZES = (21, 75, 150, 304, 700, 1500, 3000, 4700)
_NUM_CLASSES = int(np.sum(_GROUP_SIZES))
_KMAX = 500
_B = 1024
_RB = 128
_NEG_INF = float("-inf")
_LO_INIT = np.int32(-2139095042)
_HI_INIT = np.int32(2139095040)


def _group_windows():
    offs = np.cumsum([0] + list(_GROUP_SIZES))
    wins = []
    for g, size in enumerate(_GROUP_SIZES):
        off, end = int(offs[g]), int(offs[g + 1])
        ws = (off // 128) * 128
        we = min(((end + 127) // 128) * 128, _NUM_CLASSES)
        wins.append((off, end, ws, we, min(size, _KMAX)))
    return wins


_WINDOWS = _group_windows()


def _key_to_f32(m):
    ti = jnp.where(m < 0, m ^ jnp.int32(0x7FFFFFFF), m)
    return jax.lax.bitcast_convert_type(ti, jnp.float32)


def _rowsum(x, ones):
    return jax.lax.dot_general(
        x, ones, (((1,), (0,)), ((), ())), preferred_element_type=jnp.float32)


def _kl_terms(wsel, e_t, e_s, t, s, m_t, m_s, ones, rows_norm):
    w = wsel * e_t
    z_t = _rowsum(w, ones)
    s_wt = _rowsum(w * t, ones)
    s_ts = _rowsum(w * s, ones)
    z_s = _rowsum(wsel * e_s, ones)
    kl = (s_wt - m_t * z_t - s_ts) / z_t - jnp.log(z_t) + m_s + jnp.log(z_s)
    return jnp.sum(kl) * rows_norm


def _loss_body(s_ref, t_ref, o_ref):
    pid = pl.program_id(0)
    total = jnp.float32(0.0)
    big = []
    for g, (off, end, ws, we, k) in enumerate(_WINDOWS):
        size = end - off
        t = t_ref[:, ws:we]
        s = s_ref[:, ws:we]
        cols = jax.lax.broadcasted_iota(jnp.int32, t.shape, 1) + ws
        mask = (cols >= off) & (cols < end)
        tm = jnp.where(mask, t, _NEG_INF)
        sm = jnp.where(mask, s, _NEG_INF)
        ones = jnp.ones((t.shape[1], 1), jnp.float32)
        norm = jnp.float32(size / float(_NUM_CLASSES) / float(_B))
        if k == size:
            m_t = jnp.max(tm, axis=1, keepdims=True)
            m_s = jnp.max(sm, axis=1, keepdims=True)
            e_t = jnp.exp(tm - m_t)
            e_s = jnp.exp(sm - m_s)
            total = total + _kl_terms(
                jnp.float32(1.0), e_t, e_s, t, s, m_t, m_s, ones, norm)
        else:
            big.append((t, s, tm, sm, k, norm, ones))

    nbig = len(big)
    rows = big[0][0].shape[0]
    los = tuple(jnp.full((rows, 1), _LO_INIT, jnp.int32) for _ in range(nbig))
    his = tuple(jnp.full((rows, 1), _HI_INIT, jnp.int32) for _ in range(nbig))

    def body(_, carry):
        los, his = carry
        nlos, nhis = [], []
        for gi in range(nbig):
            lo, hi = los[gi], his[gi]
            mid = (lo >> 1) + (hi >> 1) + ((lo | hi) & 1)
            f_mid = _key_to_f32(mid)
            ind = jnp.where(big[gi][2] >= f_mid, 1.0, 0.0)
            cnt = _rowsum(ind, big[gi][6])
            ge = cnt >= jnp.float32(big[gi][4])
            nlos.append(jnp.where(ge, mid, lo))
            nhis.append(jnp.where(ge, hi, mid - 1))
        return tuple(nlos), tuple(nhis)

    los, his = jax.lax.fori_loop(0, 32, body, (los, his), unroll=32)

    for gi in range(nbig):
        t, s, tm, sm, k, norm, ones = big[gi]
        f_tau = _key_to_f32(los[gi])
        gt01 = jnp.where(tm > f_tau, 1.0, 0.0)
        eq01 = jnp.where(tm == f_tau, 1.0, 0.0)
        cgt = _rowsum(gt01, ones)
        ceq = _rowsum(eq01, ones)
        frac = (jnp.float32(k) - cgt) / ceq
        wsel = gt01 + frac * eq01
        m_t = jnp.max(tm, axis=1, keepdims=True)
        m_s = jnp.max(sm, axis=1, keepdims=True)
        e_t = jnp.exp(tm - m_t)
        e_s = jnp.exp(sm - m_s)
        total = total + _kl_terms(wsel, e_t, e_s, t, s, m_t, m_s, ones, norm)

    o_ref[0, 0] = jnp.where(pid == 0, total, o_ref[0, 0] + total)


@jax.jit
def kernel(logits, logits_teacher, targets):
    del targets
    out = pl.pallas_call(
        _loss_body,
        grid=(_B // _RB,),
        in_specs=[
            pl.BlockSpec((_RB, _NUM_CLASSES), lambda i: (i, 0)),
            pl.BlockSpec((_RB, _NUM_CLASSES), lambda i: (i, 0)),
        ],
        out_specs=pl.BlockSpec(memory_space=pltpu.SMEM),
        out_shape=jax.ShapeDtypeStruct((1, 1), jnp.float32),
    )(logits, logits_teacher)
    return out[0, 0]
